# Initial kernel scaffold; baseline (speedup 1.0000x reference)
import jax
import jax.numpy as jnp
from jax.experimental import pallas as pl


def kernel(feat, params):
    raise NotImplementedError("write your pallas kernel here")


# speedup vs baseline: 3.7857x; 3.7857x over previous
import numpy as np
import jax
import jax.numpy as jnp
from jax.experimental import pallas as pl
from jax.experimental.pallas import tpu as pltpu

P = 16
D = 256
E = 384
NS = 16
KS = 3
C_IN = 40
H = 512
W = 512
HP = H // P
WP = W // P
J = HP * WP
PD = C_IN * P * P
CC = 4 * C_IN
_PATHS = ('row', 'col', 'diag', 'diag_rev')


def _scan_order(mode):
    coords = []
    if mode == 'row':
        coords = [(r, c) for r in range(HP) for c in range(WP)]
    elif mode == 'col':
        coords = [(r, c) for c in range(WP) for r in range(HP)]
    elif mode == 'diag':
        for s in range(HP + WP - 1):
            for r in range(HP):
                c = s - r
                if 0 <= c < WP:
                    coords.append((r, c))
    else:
        for s in range(HP + WP - 2, -1, -1):
            for r in range(HP):
                c = s - r
                if 0 <= c < WP:
                    coords.append((r, c))
    return np.array([r * WP + c for r, c in coords])


_IDX = np.stack([_scan_order(m) for m in _PATHS])
_INV = np.stack([np.argsort(_IDX[i]) for i in range(4)])
_HIDX = np.stack([[P * i, P * i + P + 1] for i in range(HP)]).reshape(-1)


def _pos_emb():
    pos = jnp.arange(J, dtype=jnp.float32)[:, None]
    i = jnp.arange(D)
    div = jnp.exp(-np.log(10000.0) * (2.0 * (i // 2).astype(jnp.float32) / D))[None, :]
    ang = pos * div
    return jnp.where(i[None, :] % 2 == 0, jnp.sin(ang), jnp.cos(ang))


def _proj_in_kernel(p_ref, w_ref, bpe_ref, o_ref):
    o_ref[...] = (jnp.dot(p_ref[...], w_ref[...], preferred_element_type=jnp.float32)
                  + bpe_ref[...])


def _front_kernel(tok_ref, lng_ref, lnb_ref, xw_ref, xb_ref, zw_ref, zb_ref,
                  x_ref, g_ref):
    t = tok_ref[...]
    m = jnp.mean(t, axis=-1, keepdims=True)
    c = t - m
    v = jnp.mean(c * c, axis=-1, keepdims=True)
    tn = c * jax.lax.rsqrt(v + 1e-5) * lng_ref[0] + lnb_ref[0]
    x_ref[0] = jnp.dot(tn, xw_ref[0], preferred_element_type=jnp.float32) + xb_ref[0]
    zp = jnp.dot(tn, zw_ref[0], preferred_element_type=jnp.float32) + zb_ref[0]
    g_ref[0] = jax.nn.sigmoid(zp * jax.nn.sigmoid(zp))


def _softplus(x):
    return jnp.maximum(x, 0.0) + jnp.log1p(jnp.exp(-jnp.abs(x)))


def _scan_kernel(x_ref, cw_ref, cb_ref, wp_ref, bp_ref, al_ref, ro_ref, rob_ref,
                 y_ref):
    x = x_ref[0]
    zrow = jnp.zeros((1, E), jnp.float32)
    xm = jnp.concatenate([zrow, x[:-1]], axis=0)
    xp = jnp.concatenate([x[1:], zrow], axis=0)
    w = cw_ref[0]
    xc = w[0:1] * xm + w[1:2] * x + w[2:3] * xp + cb_ref[0]
    bcdu = jnp.dot(xc, wp_ref[0], preferred_element_type=jnp.float32) + bp_ref[0]
    bt = bcdu[:, 0:NS]
    ct = bcdu[:, NS:2 * NS]
    dl = bcdu[:, 2 * NS:3 * NS]
    u = bcdu[:, 3 * NS:4 * NS]
    delta = _softplus(dl)
    a_coef = -_softplus(al_ref[0])
    ab = jnp.exp(delta * a_coef)
    bu = (ab - 1.0) / (a_coef + 1e-6) * bt * u
    a = ab
    b = bu
    s = 1
    while s < J:
        a_prev = jnp.concatenate([jnp.ones((s, NS), jnp.float32), a[:J - s]], axis=0)
        b_prev = jnp.concatenate([jnp.zeros((s, NS), jnp.float32), b[:J - s]], axis=0)
        b = a * b_prev + b
        a = a * a_prev
        s *= 2
    y_ref[0] = jnp.dot(b * ct, ro_ref[0], preferred_element_type=jnp.float32) + rob_ref[0]


def _back_kernel(y_ref, g_ref, tok_ref, pw_ref, pb_ref, ow_ref, ob_ref, pf_ref):
    yg = y_ref[0] * g_ref[0]
    o = jnp.dot(yg, pw_ref[0], preferred_element_type=jnp.float32) + pb_ref[0] \
        + tok_ref[...]
    pf_ref[0] = jnp.dot(o, ow_ref[...], preferred_element_type=jnp.float32) + ob_ref[...]


def _conv_kernel(x_ref, h_ref, wf_ref, fb_ref, o_ref):
    top = h_ref[0, 0:1]
    bot = h_ref[0, 1:2]
    xf = jnp.concatenate([top, x_ref[0], bot], axis=0)
    zcol = jnp.zeros((P, 1, CC), jnp.float32)
    acc = jnp.zeros((P * W, C_IN), jnp.float32)
    for di in range(3):
        rows = xf[di:di + P]
        for dj in range(3):
            if dj == 0:
                xs = jnp.concatenate([zcol, rows[:, :W - 1]], axis=1)
            elif dj == 1:
                xs = rows
            else:
                xs = jnp.concatenate([rows[:, 1:], zcol], axis=1)
            acc = acc + jnp.dot(xs.reshape(P * W, CC), wf_ref[di * 3 + dj],
                                preferred_element_type=jnp.float32)
    o_ref[0] = (acc + fb_ref[...]).reshape(P, W, C_IN)


def kernel(feat, params):
    B = feat.shape[0]
    M = B * J

    patches = (feat.reshape(B, C_IN, HP, P, WP, P)
               .transpose(0, 2, 4, 1, 3, 5).reshape(M, PD))
    bpe = jnp.tile(_pos_emb() + params['proj_in_b'][None, :], (B, 1))

    tokens = pl.pallas_call(
        _proj_in_kernel,
        grid=(M // 128,),
        in_specs=[pl.BlockSpec((128, PD), lambda i: (i, 0)),
                  pl.BlockSpec((PD, D), lambda i: (0, 0)),
                  pl.BlockSpec((128, D), lambda i: (i, 0))],
        out_specs=pl.BlockSpec((128, D), lambda i: (i, 0)),
        out_shape=jax.ShapeDtypeStruct((M, D), jnp.float32),
        compiler_params=pltpu.CompilerParams(dimension_semantics=("parallel",)),
        name="proj_in")(patches, params['proj_in_W'].T, bpe)

    MT2 = 256
    x4, g4 = pl.pallas_call(
        _front_kernel,
        grid=(4, M // MT2),
        in_specs=[pl.BlockSpec((MT2, D), lambda p, i: (i, 0)),
                  pl.BlockSpec((1, 1, D), lambda p, i: (p, 0, 0)),
                  pl.BlockSpec((1, 1, D), lambda p, i: (p, 0, 0)),
                  pl.BlockSpec((1, D, E), lambda p, i: (p, 0, 0)),
                  pl.BlockSpec((1, 1, E), lambda p, i: (p, 0, 0)),
                  pl.BlockSpec((1, D, E), lambda p, i: (p, 0, 0)),
                  pl.BlockSpec((1, 1, E), lambda p, i: (p, 0, 0))],
        out_specs=[pl.BlockSpec((1, MT2, E), lambda p, i: (p, i, 0)),
                   pl.BlockSpec((1, MT2, E), lambda p, i: (p, i, 0))],
        out_shape=[jax.ShapeDtypeStruct((4, M, E), jnp.float32),
                   jax.ShapeDtypeStruct((4, M, E), jnp.float32)],
        compiler_params=pltpu.CompilerParams(
            dimension_semantics=("parallel", "parallel")),
        name="vim_front")(
            tokens,
            params['ln_g'][:, None, :], params['ln_b'][:, None, :],
            params['xW'].transpose(0, 2, 1), params['xb'][:, None, :],
            params['zW'].transpose(0, 2, 1), params['zb'][:, None, :])

    idx = jnp.asarray(_IDX)
    inv = jnp.asarray(_INV)
    x4r = x4.reshape(4, B, J, E)
    xperm = jnp.take_along_axis(x4r, idx[:, None, :, None], axis=2)
    x16 = jnp.stack([xperm, xperm[:, :, ::-1]], axis=1).reshape(8 * B, J, E)

    cw8 = (jnp.stack([params['convf_W'][:, :, 0, :], params['convb_W'][:, :, 0, :]],
                     axis=1).reshape(8, E, KS).transpose(0, 2, 1))
    cb8 = jnp.stack([params['convf_b'], params['convb_b']], axis=1).reshape(8, 1, E)
    wf_cat = jnp.concatenate([params['Bf_W'], params['Cf_W'], params['Df_W'],
                              params['u_W']], axis=1)
    wb_cat = jnp.concatenate([params['Bb_W'], params['Cb_W'], params['Db_W'],
                              params['u_W']], axis=1)
    wp8 = jnp.stack([wf_cat, wb_cat], axis=1).reshape(8, 4 * NS, E).transpose(0, 2, 1)
    bf_cat = jnp.concatenate([params['Bf_b'], params['Cf_b'], params['Df_b'],
                              params['u_b']], axis=1)
    bb_cat = jnp.concatenate([params['Bb_b'], params['Cb_b'], params['Db_b'],
                              params['u_b']], axis=1)
    bp8 = jnp.stack([bf_cat, bb_cat], axis=1).reshape(8, 1, 4 * NS)
    al8 = jnp.repeat(params['A_log'][:, None, :], 2, axis=1).reshape(8, 1, NS)
    ro8 = jnp.repeat(params['ro_W'].transpose(0, 2, 1)[:, None], 2, axis=1
                     ).reshape(8, NS, E)
    rob8 = jnp.repeat(params['ro_b'][:, None, :], 2, axis=1).reshape(8, 1, E)

    nb = B
    y16 = pl.pallas_call(
        _scan_kernel,
        grid=(8 * B,),
        in_specs=[pl.BlockSpec((1, J, E), lambda i: (i, 0, 0)),
                  pl.BlockSpec((1, KS, E), lambda i: (i // nb, 0, 0)),
                  pl.BlockSpec((1, 1, E), lambda i: (i // nb, 0, 0)),
                  pl.BlockSpec((1, E, 4 * NS), lambda i: (i // nb, 0, 0)),
                  pl.BlockSpec((1, 1, 4 * NS), lambda i: (i // nb, 0, 0)),
                  pl.BlockSpec((1, 1, NS), lambda i: (i // nb, 0, 0)),
                  pl.BlockSpec((1, NS, E), lambda i: (i // nb, 0, 0)),
                  pl.BlockSpec((1, 1, E), lambda i: (i // nb, 0, 0))],
        out_specs=pl.BlockSpec((1, J, E), lambda i: (i, 0, 0)),
        out_shape=jax.ShapeDtypeStruct((8 * B, J, E), jnp.float32),
        compiler_params=pltpu.CompilerParams(dimension_semantics=("parallel",)),
        name="ssm_scan")(x16, cw8, cb8, wp8, bp8, al8, ro8, rob8)

    y = y16.reshape(4, 2, B, J, E)
    ysum = y[:, 0] + y[:, 1, :, ::-1]
    ytok = jnp.take_along_axis(ysum, inv[:, None, :, None], axis=2).reshape(4, M, E)

    MT4 = 128
    pf = pl.pallas_call(
        _back_kernel,
        grid=(4, M // MT4),
        in_specs=[pl.BlockSpec((1, MT4, E), lambda p, i: (p, i, 0)),
                  pl.BlockSpec((1, MT4, E), lambda p, i: (p, i, 0)),
                  pl.BlockSpec((MT4, D), lambda p, i: (i, 0)),
                  pl.BlockSpec((1, E, D), lambda p, i: (p, 0, 0)),
                  pl.BlockSpec((1, 1, D), lambda p, i: (p, 0, 0)),
                  pl.BlockSpec((D, PD), lambda p, i: (0, 0)),
                  pl.BlockSpec((1, PD), lambda p, i: (0, 0))],
        out_specs=pl.BlockSpec((1, MT4, PD), lambda p, i: (p, i, 0)),
        out_shape=jax.ShapeDtypeStruct((4, M, PD), jnp.float32),
        compiler_params=pltpu.CompilerParams(
            dimension_semantics=("parallel", "parallel")),
        name="vim_back")(
            ytok, g4, tokens,
            params['po_W'].transpose(0, 2, 1), params['po_b'][:, None, :],
            params['out_W'].T, params['out_b'][None, :])

    cat = (pf.reshape(4, B, HP, WP, C_IN, P, P)
           .transpose(1, 2, 5, 3, 6, 0, 4).reshape(B, H, W, CC))
    catp = jnp.pad(cat, ((0, 0), (1, 1), (0, 0), (0, 0)))
    halo = catp[:, jnp.asarray(_HIDX)].reshape(B * HP, 2, W, CC)

    outc = pl.pallas_call(
        _conv_kernel,
        grid=(B * HP,),
        in_specs=[pl.BlockSpec((1, P, W, CC), lambda i: (i, 0, 0, 0)),
                  pl.BlockSpec((1, 2, W, CC), lambda i: (i, 0, 0, 0)),
                  pl.BlockSpec((9, CC, C_IN), lambda i: (0, 0, 0)),
                  pl.BlockSpec((1, C_IN), lambda i: (0, 0))],
        out_specs=pl.BlockSpec((1, P, W, C_IN), lambda i: (i, 0, 0, 0)),
        out_shape=jax.ShapeDtypeStruct((B * HP, P, W, C_IN), jnp.float32),
        compiler_params=pltpu.CompilerParams(dimension_semantics=("parallel",)),
        name="fuse_conv")(
            cat.reshape(B * HP, P, W, CC), halo,
            params['fuse_W'].transpose(2, 3, 1, 0).reshape(9, CC, C_IN),
            params['fuse_b'][None, :])

    return outc.reshape(B, H, W, C_IN).transpose(0, 3, 1, 2)


# speedup vs baseline: 5.4977x; 1.4522x over previous
import numpy as np
import jax
import jax.numpy as jnp
from jax.experimental import pallas as pl
from jax.experimental.pallas import tpu as pltpu

P = 16
D = 256
E = 384
NS = 16
KS = 3
C_IN = 40
H = 512
W = 512
HP = H // P
WP = W // P
J = HP * WP
PD = C_IN * P * P
CC = 4 * C_IN
_PATHS = ('row', 'col', 'diag', 'diag_rev')


def _scan_order(mode):
    coords = []
    if mode == 'row':
        coords = [(r, c) for r in range(HP) for c in range(WP)]
    elif mode == 'col':
        coords = [(r, c) for c in range(WP) for r in range(HP)]
    elif mode == 'diag':
        for s in range(HP + WP - 1):
            for r in range(HP):
                c = s - r
                if 0 <= c < WP:
                    coords.append((r, c))
    else:
        for s in range(HP + WP - 2, -1, -1):
            for r in range(HP):
                c = s - r
                if 0 <= c < WP:
                    coords.append((r, c))
    return np.array([r * WP + c for r, c in coords])


_IDX = np.stack([_scan_order(m) for m in _PATHS])
_INV = np.stack([np.argsort(_IDX[i]) for i in range(4)])


def _pos_emb():
    pos = jnp.arange(J, dtype=jnp.float32)[:, None]
    i = jnp.arange(D)
    div = jnp.exp(-np.log(10000.0) * (2.0 * (i // 2).astype(jnp.float32) / D))[None, :]
    ang = pos * div
    return jnp.where(i[None, :] % 2 == 0, jnp.sin(ang), jnp.cos(ang))


def _proj_in_kernel(p_ref, w_ref, bpe_ref, o_ref):
    o_ref[...] = (jnp.dot(p_ref[...].astype(jnp.bfloat16), w_ref[...],
                          preferred_element_type=jnp.float32)
                  + bpe_ref[...])


def _front_kernel(tok_ref, lng_ref, lnb_ref, xw_ref, xb_ref, zw_ref, zb_ref,
                  x_ref, g_ref):
    t = tok_ref[...]
    m = jnp.mean(t, axis=-1, keepdims=True)
    c = t - m
    v = jnp.mean(c * c, axis=-1, keepdims=True)
    tn = c * jax.lax.rsqrt(v + 1e-5) * lng_ref[0] + lnb_ref[0]
    x_ref[0] = jnp.dot(tn, xw_ref[0], preferred_element_type=jnp.float32) + xb_ref[0]
    zp = jnp.dot(tn, zw_ref[0], preferred_element_type=jnp.float32) + zb_ref[0]
    g_ref[0] = jax.nn.sigmoid(zp * jax.nn.sigmoid(zp))


def _softplus(x):
    return jnp.maximum(x, 0.0) + jnp.log1p(jnp.exp(-jnp.abs(x)))


def _ssm_coeffs(bcdu, a_coef):
    bt = bcdu[:, 0:NS]
    ct = bcdu[:, NS:2 * NS]
    dl = bcdu[:, 2 * NS:3 * NS]
    u = bcdu[:, 3 * NS:4 * NS]
    delta = _softplus(dl)
    ab = jnp.exp(delta * a_coef)
    bu = (ab - 1.0) / (a_coef + 1e-6) * bt * u
    return ab, bu, ct


def _scan_kernel(x_ref, cw_ref, cb_ref, wp_ref, bp_ref, al_ref, ro_ref, rob_ref,
                 y_ref):
    x = x_ref[0]
    zrow = jnp.zeros((1, E), jnp.float32)
    xm = jnp.concatenate([zrow, x[:-1]], axis=0)
    xp = jnp.concatenate([x[1:], zrow], axis=0)
    wf = cw_ref[0, 0]
    wb = cw_ref[0, 1]
    xcf = wf[0:1] * xm + wf[1:2] * x + wf[2:3] * xp + cb_ref[0, 0]
    xcb = wb[2:3] * xm + wb[1:2] * x + wb[0:1] * xp + cb_ref[0, 1]
    bcduf = jnp.dot(xcf, wp_ref[0, 0], preferred_element_type=jnp.float32) + bp_ref[0, 0]
    bcdub = jnp.dot(xcb, wp_ref[0, 1], preferred_element_type=jnp.float32) + bp_ref[0, 1]
    a_coef = -_softplus(al_ref[0])

    af, bf, ctf = _ssm_coeffs(bcduf, a_coef)
    ab_, bb_, ctb = _ssm_coeffs(bcdub, a_coef)

    s = 1
    while s < J:
        a_prev = jnp.concatenate([jnp.ones((s, NS), jnp.float32), af[:J - s]], axis=0)
        b_prev = jnp.concatenate([jnp.zeros((s, NS), jnp.float32), bf[:J - s]], axis=0)
        bf = af * b_prev + bf
        af = af * a_prev
        s *= 2
    s = 1
    while s < J:
        a_nxt = jnp.concatenate([ab_[s:], jnp.ones((s, NS), jnp.float32)], axis=0)
        b_nxt = jnp.concatenate([bb_[s:], jnp.zeros((s, NS), jnp.float32)], axis=0)
        bb_ = ab_ * b_nxt + bb_
        ab_ = ab_ * a_nxt
        s *= 2

    hc = bf * ctf + bb_ * ctb
    y_ref[0] = (jnp.dot(hc, ro_ref[0], preferred_element_type=jnp.float32)
                + 2.0 * rob_ref[0])


def _back_kernel(y_ref, g_ref, tok_ref, pw_ref, pb_ref, ow_ref, ob_ref, pf_ref):
    yg = y_ref[0] * g_ref[0]
    o = jnp.dot(yg, pw_ref[0], preferred_element_type=jnp.float32) + pb_ref[0] \
        + tok_ref[...]
    pf = jnp.dot(o.astype(jnp.bfloat16), ow_ref[...],
                 preferred_element_type=jnp.float32) + ob_ref[...]
    pf_ref[0] = pf.astype(jnp.bfloat16)


def _conv_kernel(x_ref, h_ref, wf_ref, fb_ref, o_ref):
    top = h_ref[0, 0:1]
    bot = h_ref[0, 1:2]
    xf = jnp.concatenate([top, x_ref[0], bot], axis=0)
    zc = jnp.zeros((P, 1, C_IN), jnp.float32)
    acc = jnp.zeros((P, W, C_IN), jnp.float32)
    for di in range(3):
        lhs = xf[di:di + P].reshape(P * W, CC)
        yv = jnp.dot(lhs, wf_ref[di], preferred_element_type=jnp.float32
                     ).reshape(P, W, 3 * C_IN)
        y0 = yv[:, :, 0:C_IN]
        y1 = yv[:, :, C_IN:2 * C_IN]
        y2 = yv[:, :, 2 * C_IN:3 * C_IN]
        acc = acc + y1
        acc = acc + jnp.concatenate([zc, y0[:, :W - 1]], axis=1)
        acc = acc + jnp.concatenate([y2[:, 1:], zc], axis=1)
    o_ref[0] = acc + fb_ref[...]


def kernel(feat, params):
    B = feat.shape[0]
    M = B * J

    patches = (feat.reshape(B, C_IN, HP, P, WP, P)
               .transpose(0, 2, 4, 1, 3, 5).reshape(M, PD))
    bpe = jnp.tile(_pos_emb() + params['proj_in_b'][None, :], (B, 1))

    tokens = pl.pallas_call(
        _proj_in_kernel,
        grid=(M // 256,),
        in_specs=[pl.BlockSpec((256, PD), lambda i: (i, 0)),
                  pl.BlockSpec((PD, D), lambda i: (0, 0)),
                  pl.BlockSpec((256, D), lambda i: (i, 0))],
        out_specs=pl.BlockSpec((256, D), lambda i: (i, 0)),
        out_shape=jax.ShapeDtypeStruct((M, D), jnp.float32),
        compiler_params=pltpu.CompilerParams(dimension_semantics=("parallel",)),
        name="proj_in")(patches, params['proj_in_W'].T.astype(jnp.bfloat16), bpe)

    MT2 = 256
    x4, g4 = pl.pallas_call(
        _front_kernel,
        grid=(4, M // MT2),
        in_specs=[pl.BlockSpec((MT2, D), lambda p, i: (i, 0)),
                  pl.BlockSpec((1, 1, D), lambda p, i: (p, 0, 0)),
                  pl.BlockSpec((1, 1, D), lambda p, i: (p, 0, 0)),
                  pl.BlockSpec((1, D, E), lambda p, i: (p, 0, 0)),
                  pl.BlockSpec((1, 1, E), lambda p, i: (p, 0, 0)),
                  pl.BlockSpec((1, D, E), lambda p, i: (p, 0, 0)),
                  pl.BlockSpec((1, 1, E), lambda p, i: (p, 0, 0))],
        out_specs=[pl.BlockSpec((1, MT2, E), lambda p, i: (p, i, 0)),
                   pl.BlockSpec((1, MT2, E), lambda p, i: (p, i, 0))],
        out_shape=[jax.ShapeDtypeStruct((4, M, E), jnp.float32),
                   jax.ShapeDtypeStruct((4, M, E), jnp.float32)],
        compiler_params=pltpu.CompilerParams(
            dimension_semantics=("parallel", "parallel")),
        name="vim_front")(
            tokens,
            params['ln_g'][:, None, :], params['ln_b'][:, None, :],
            params['xW'].transpose(0, 2, 1), params['xb'][:, None, :],
            params['zW'].transpose(0, 2, 1), params['zb'][:, None, :])

    idx = jnp.asarray(_IDX)
    inv = jnp.asarray(_INV)
    x4r = x4.reshape(4, B, J, E)
    x8 = jnp.take_along_axis(x4r, idx[:, None, :, None], axis=2).reshape(4 * B, J, E)

    cw2 = jnp.stack([params['convf_W'][:, :, 0, :], params['convb_W'][:, :, 0, :]],
                    axis=1).transpose(0, 1, 3, 2)
    cb2 = jnp.stack([params['convf_b'], params['convb_b']], axis=1)[:, :, None, :]
    wf_cat = jnp.concatenate([params['Bf_W'], params['Cf_W'], params['Df_W'],
                              params['u_W']], axis=1)
    wb_cat = jnp.concatenate([params['Bb_W'], params['Cb_W'], params['Db_W'],
                              params['u_W']], axis=1)
    wp2 = jnp.stack([wf_cat, wb_cat], axis=1).transpose(0, 1, 3, 2)
    bf_cat = jnp.concatenate([params['Bf_b'], params['Cf_b'], params['Df_b'],
                              params['u_b']], axis=1)
    bb_cat = jnp.concatenate([params['Bb_b'], params['Cb_b'], params['Db_b'],
                              params['u_b']], axis=1)
    bp2 = jnp.stack([bf_cat, bb_cat], axis=1)[:, :, None, :]
    al4 = params['A_log'][:, None, :]
    ro4 = params['ro_W'].transpose(0, 2, 1)
    rob4 = params['ro_b'][:, None, :]

    nb = B
    ysum = pl.pallas_call(
        _scan_kernel,
        grid=(4 * B,),
        in_specs=[pl.BlockSpec((1, J, E), lambda i: (i, 0, 0)),
                  pl.BlockSpec((1, 2, KS, E), lambda i: (i // nb, 0, 0, 0)),
                  pl.BlockSpec((1, 2, 1, E), lambda i: (i // nb, 0, 0, 0)),
                  pl.BlockSpec((1, 2, E, 4 * NS), lambda i: (i // nb, 0, 0, 0)),
                  pl.BlockSpec((1, 2, 1, 4 * NS), lambda i: (i // nb, 0, 0, 0)),
                  pl.BlockSpec((1, 1, NS), lambda i: (i // nb, 0, 0)),
                  pl.BlockSpec((1, NS, E), lambda i: (i // nb, 0, 0)),
                  pl.BlockSpec((1, 1, E), lambda i: (i // nb, 0, 0))],
        out_specs=pl.BlockSpec((1, J, E), lambda i: (i, 0, 0)),
        out_shape=jax.ShapeDtypeStruct((4 * B, J, E), jnp.float32),
        compiler_params=pltpu.CompilerParams(dimension_semantics=("parallel",)),
        name="ssm_scan")(x8, cw2, cb2, wp2, bp2, al4, ro4, rob4)

    ytok = jnp.take_along_axis(ysum.reshape(4, B, J, E), inv[:, None, :, None],
                               axis=2).reshape(4, M, E)

    MT4 = 256
    pf = pl.pallas_call(
        _back_kernel,
        grid=(4, M // MT4),
        in_specs=[pl.BlockSpec((1, MT4, E), lambda p, i: (p, i, 0)),
                  pl.BlockSpec((1, MT4, E), lambda p, i: (p, i, 0)),
                  pl.BlockSpec((MT4, D), lambda p, i: (i, 0)),
                  pl.BlockSpec((1, E, D), lambda p, i: (p, 0, 0)),
                  pl.BlockSpec((1, 1, D), lambda p, i: (p, 0, 0)),
                  pl.BlockSpec((D, PD), lambda p, i: (0, 0)),
                  pl.BlockSpec((1, PD), lambda p, i: (0, 0))],
        out_specs=pl.BlockSpec((1, MT4, PD), lambda p, i: (p, i, 0)),
        out_shape=jax.ShapeDtypeStruct((4, M, PD), jnp.bfloat16),
        compiler_params=pltpu.CompilerParams(
            dimension_semantics=("parallel", "parallel")),
        name="vim_back")(
            ytok, g4, tokens,
            params['po_W'].transpose(0, 2, 1), params['po_b'][:, None, :],
            params['out_W'].T.astype(jnp.bfloat16), params['out_b'][None, :])

    cat = (pf.reshape(4, B, HP, WP, C_IN, P, P)
           .transpose(1, 2, 5, 3, 6, 0, 4).reshape(B, H, W, CC))
    zrow1 = jnp.zeros((B, 1, W, CC), jnp.bfloat16)
    tops = jnp.concatenate([zrow1, cat[:, P - 1:H - 1:P]], axis=1)
    bots = jnp.concatenate([cat[:, P::P], zrow1], axis=1)
    halo = jnp.stack([tops, bots], axis=2).reshape(B * HP, 2, W, CC)

    wfr = (params['fuse_W'].transpose(2, 3, 1, 0)
           .transpose(0, 2, 1, 3).reshape(3, CC, 3 * C_IN).astype(jnp.bfloat16))

    outc = pl.pallas_call(
        _conv_kernel,
        grid=(B * HP,),
        in_specs=[pl.BlockSpec((1, P, W, CC), lambda i: (i, 0, 0, 0)),
                  pl.BlockSpec((1, 2, W, CC), lambda i: (i, 0, 0, 0)),
                  pl.BlockSpec((3, CC, 3 * C_IN), lambda i: (0, 0, 0)),
                  pl.BlockSpec((1, C_IN), lambda i: (0, 0))],
        out_specs=pl.BlockSpec((1, P, W, C_IN), lambda i: (i, 0, 0, 0)),
        out_shape=jax.ShapeDtypeStruct((B * HP, P, W, C_IN), jnp.float32),
        compiler_params=pltpu.CompilerParams(dimension_semantics=("parallel",)),
        name="fuse_conv")(
            cat.reshape(B * HP, P, W, CC), halo, wfr, params['fuse_b'][None, :])

    return outc.reshape(B, H, W, C_IN).transpose(0, 3, 1, 2)


# speedup vs baseline: 5.5826x; 1.0154x over previous
import numpy as np
import jax
import jax.numpy as jnp
from jax.experimental import pallas as pl
from jax.experimental.pallas import tpu as pltpu

P = 16
D = 256
E = 384
NS = 16
KS = 3
C_IN = 40
H = 512
W = 512
HP = H // P
WP = W // P
J = HP * WP
PD = C_IN * P * P
CC = 4 * C_IN
_PATHS = ('row', 'col', 'diag', 'diag_rev')


def _scan_order(mode):
    coords = []
    if mode == 'row':
        coords = [(r, c) for r in range(HP) for c in range(WP)]
    elif mode == 'col':
        coords = [(r, c) for c in range(WP) for r in range(HP)]
    elif mode == 'diag':
        for s in range(HP + WP - 1):
            for r in range(HP):
                c = s - r
                if 0 <= c < WP:
                    coords.append((r, c))
    else:
        for s in range(HP + WP - 2, -1, -1):
            for r in range(HP):
                c = s - r
                if 0 <= c < WP:
                    coords.append((r, c))
    return np.array([r * WP + c for r, c in coords])


_IDX = np.stack([_scan_order(m) for m in _PATHS])
_INV = np.stack([np.argsort(_IDX[i]) for i in range(4)])


def _pos_emb():
    pos = jnp.arange(J, dtype=jnp.float32)[:, None]
    i = jnp.arange(D)
    div = jnp.exp(-np.log(10000.0) * (2.0 * (i // 2).astype(jnp.float32) / D))[None, :]
    ang = pos * div
    return jnp.where(i[None, :] % 2 == 0, jnp.sin(ang), jnp.cos(ang))


def _proj_in_kernel(p_ref, w_ref, bpe_ref, o_ref):
    o_ref[...] = (jnp.dot(p_ref[...].astype(jnp.bfloat16), w_ref[...],
                          preferred_element_type=jnp.float32)
                  + bpe_ref[...])


def _front_kernel(tok_ref, lng_ref, lnb_ref, xw_ref, xb_ref, zw_ref, zb_ref,
                  x_ref, g_ref):
    t = tok_ref[...]
    m = jnp.mean(t, axis=-1, keepdims=True)
    c = t - m
    v = jnp.mean(c * c, axis=-1, keepdims=True)
    tn = c * jax.lax.rsqrt(v + 1e-5) * lng_ref[0] + lnb_ref[0]
    x_ref[0] = jnp.dot(tn, xw_ref[0], preferred_element_type=jnp.float32) + xb_ref[0]
    zp = jnp.dot(tn, zw_ref[0], preferred_element_type=jnp.float32) + zb_ref[0]
    g_ref[0] = jax.nn.sigmoid(zp * jax.nn.sigmoid(zp))


def _softplus(x):
    return jnp.maximum(x, 0.0) + jnp.log1p(jnp.exp(-jnp.abs(x)))


def _ssm_coeffs(bcdu, a_coef):
    bt = bcdu[:, 0:NS]
    ct = bcdu[:, NS:2 * NS]
    dl = bcdu[:, 2 * NS:3 * NS]
    u = bcdu[:, 3 * NS:4 * NS]
    delta = _softplus(dl)
    ab = jnp.exp(delta * a_coef)
    bu = (ab - 1.0) / (a_coef + 1e-6) * bt * u
    return ab, bu, ct


def _scan_kernel(x_ref, cw_ref, cb_ref, wp_ref, bp_ref, al_ref, ro_ref, rob_ref,
                 y_ref):
    x = x_ref[0]
    zrow = jnp.zeros((1, E), jnp.float32)
    xm = jnp.concatenate([zrow, x[:-1]], axis=0)
    xp = jnp.concatenate([x[1:], zrow], axis=0)
    wf = cw_ref[0, 0]
    wb = cw_ref[0, 1]
    xcf = wf[0:1] * xm + wf[1:2] * x + wf[2:3] * xp + cb_ref[0, 0]
    xcb = wb[2:3] * xm + wb[1:2] * x + wb[0:1] * xp + cb_ref[0, 1]
    bcduf = jnp.dot(xcf, wp_ref[0, 0], preferred_element_type=jnp.float32) + bp_ref[0, 0]
    bcdub = jnp.dot(xcb, wp_ref[0, 1], preferred_element_type=jnp.float32) + bp_ref[0, 1]
    a_coef = -_softplus(al_ref[0])

    af, bf, ctf = _ssm_coeffs(bcduf, a_coef)
    ab_, bb_, ctb = _ssm_coeffs(bcdub, a_coef)

    s = 1
    while s < J:
        a_prev = jnp.concatenate([jnp.ones((s, NS), jnp.float32), af[:J - s]], axis=0)
        b_prev = jnp.concatenate([jnp.zeros((s, NS), jnp.float32), bf[:J - s]], axis=0)
        bf = af * b_prev + bf
        af = af * a_prev
        s *= 2
    s = 1
    while s < J:
        a_nxt = jnp.concatenate([ab_[s:], jnp.ones((s, NS), jnp.float32)], axis=0)
        b_nxt = jnp.concatenate([bb_[s:], jnp.zeros((s, NS), jnp.float32)], axis=0)
        bb_ = ab_ * b_nxt + bb_
        ab_ = ab_ * a_nxt
        s *= 2

    hc = bf * ctf + bb_ * ctb
    y_ref[0] = (jnp.dot(hc, ro_ref[0], preferred_element_type=jnp.float32)
                + 2.0 * rob_ref[0])


def _back_kernel(y_ref, g_ref, tok_ref, pw_ref, pb_ref, ow_ref, ob_ref, pf_ref):
    yg = y_ref[0] * g_ref[0]
    o = jnp.dot(yg, pw_ref[0], preferred_element_type=jnp.float32) + pb_ref[0] \
        + tok_ref[...]
    pf = jnp.dot(o.astype(jnp.bfloat16), ow_ref[...],
                 preferred_element_type=jnp.float32) + ob_ref[...]
    pf_ref[0] = pf.astype(jnp.bfloat16)


def _conv_kernel(x_ref, h_ref, wf_ref, fb_ref, o_ref):
    top = h_ref[0, 0:W]
    bot = h_ref[0, W:2 * W]
    xf = jnp.concatenate([top, x_ref[0], bot], axis=0)
    widx = jax.lax.broadcasted_iota(jnp.int32, (P * W, C_IN), 0) % W
    zrow = jnp.zeros((1, C_IN), jnp.float32)
    acc = jnp.zeros((P * W, C_IN), jnp.float32)
    for di in range(3):
        lhs = xf[di * W:di * W + P * W]
        yv = jnp.dot(lhs, wf_ref[di], preferred_element_type=jnp.float32)
        y0 = yv[:, 0:C_IN]
        y1 = yv[:, 128:128 + C_IN]
        y2 = yv[:, 256:256 + C_IN]
        acc = acc + y1
        sd = jnp.concatenate([zrow, y0[:P * W - 1]], axis=0)
        acc = acc + jnp.where(widx > 0, sd, 0.0)
        su = jnp.concatenate([y2[1:], zrow], axis=0)
        acc = acc + jnp.where(widx < W - 1, su, 0.0)
    o_ref[0] = acc + fb_ref[...]


def kernel(feat, params):
    B = feat.shape[0]
    M = B * J

    patches = (feat.reshape(B, C_IN, HP, P, WP, P)
               .transpose(0, 2, 4, 1, 3, 5).reshape(M, PD))
    bpe = jnp.tile(_pos_emb() + params['proj_in_b'][None, :], (B, 1))

    tokens = pl.pallas_call(
        _proj_in_kernel,
        grid=(M // 256,),
        in_specs=[pl.BlockSpec((256, PD), lambda i: (i, 0)),
                  pl.BlockSpec((PD, D), lambda i: (0, 0)),
                  pl.BlockSpec((256, D), lambda i: (i, 0))],
        out_specs=pl.BlockSpec((256, D), lambda i: (i, 0)),
        out_shape=jax.ShapeDtypeStruct((M, D), jnp.float32),
        compiler_params=pltpu.CompilerParams(dimension_semantics=("parallel",)),
        name="proj_in")(patches, params['proj_in_W'].T.astype(jnp.bfloat16), bpe)

    MT2 = 256
    x4, g4 = pl.pallas_call(
        _front_kernel,
        grid=(4, M // MT2),
        in_specs=[pl.BlockSpec((MT2, D), lambda p, i: (i, 0)),
                  pl.BlockSpec((1, 1, D), lambda p, i: (p, 0, 0)),
                  pl.BlockSpec((1, 1, D), lambda p, i: (p, 0, 0)),
                  pl.BlockSpec((1, D, E), lambda p, i: (p, 0, 0)),
                  pl.BlockSpec((1, 1, E), lambda p, i: (p, 0, 0)),
                  pl.BlockSpec((1, D, E), lambda p, i: (p, 0, 0)),
                  pl.BlockSpec((1, 1, E), lambda p, i: (p, 0, 0))],
        out_specs=[pl.BlockSpec((1, MT2, E), lambda p, i: (p, i, 0)),
                   pl.BlockSpec((1, MT2, E), lambda p, i: (p, i, 0))],
        out_shape=[jax.ShapeDtypeStruct((4, M, E), jnp.float32),
                   jax.ShapeDtypeStruct((4, M, E), jnp.float32)],
        compiler_params=pltpu.CompilerParams(
            dimension_semantics=("parallel", "parallel")),
        name="vim_front")(
            tokens,
            params['ln_g'][:, None, :], params['ln_b'][:, None, :],
            params['xW'].transpose(0, 2, 1), params['xb'][:, None, :],
            params['zW'].transpose(0, 2, 1), params['zb'][:, None, :])

    idx = jnp.asarray(_IDX)
    inv = jnp.asarray(_INV)
    x4r = x4.reshape(4, B, J, E)
    x8 = jnp.take_along_axis(x4r, idx[:, None, :, None], axis=2).reshape(4 * B, J, E)

    cw2 = jnp.stack([params['convf_W'][:, :, 0, :], params['convb_W'][:, :, 0, :]],
                    axis=1).transpose(0, 1, 3, 2)
    cb2 = jnp.stack([params['convf_b'], params['convb_b']], axis=1)[:, :, None, :]
    wf_cat = jnp.concatenate([params['Bf_W'], params['Cf_W'], params['Df_W'],
                              params['u_W']], axis=1)
    wb_cat = jnp.concatenate([params['Bb_W'], params['Cb_W'], params['Db_W'],
                              params['u_W']], axis=1)
    wp2 = jnp.stack([wf_cat, wb_cat], axis=1).transpose(0, 1, 3, 2)
    bf_cat = jnp.concatenate([params['Bf_b'], params['Cf_b'], params['Df_b'],
                              params['u_b']], axis=1)
    bb_cat = jnp.concatenate([params['Bb_b'], params['Cb_b'], params['Db_b'],
                              params['u_b']], axis=1)
    bp2 = jnp.stack([bf_cat, bb_cat], axis=1)[:, :, None, :]
    al4 = params['A_log'][:, None, :]
    ro4 = params['ro_W'].transpose(0, 2, 1)
    rob4 = params['ro_b'][:, None, :]

    nb = B
    ysum = pl.pallas_call(
        _scan_kernel,
        grid=(4 * B,),
        in_specs=[pl.BlockSpec((1, J, E), lambda i: (i, 0, 0)),
                  pl.BlockSpec((1, 2, KS, E), lambda i: (i // nb, 0, 0, 0)),
                  pl.BlockSpec((1, 2, 1, E), lambda i: (i // nb, 0, 0, 0)),
                  pl.BlockSpec((1, 2, E, 4 * NS), lambda i: (i // nb, 0, 0, 0)),
                  pl.BlockSpec((1, 2, 1, 4 * NS), lambda i: (i // nb, 0, 0, 0)),
                  pl.BlockSpec((1, 1, NS), lambda i: (i // nb, 0, 0)),
                  pl.BlockSpec((1, NS, E), lambda i: (i // nb, 0, 0)),
                  pl.BlockSpec((1, 1, E), lambda i: (i // nb, 0, 0))],
        out_specs=pl.BlockSpec((1, J, E), lambda i: (i, 0, 0)),
        out_shape=jax.ShapeDtypeStruct((4 * B, J, E), jnp.float32),
        compiler_params=pltpu.CompilerParams(dimension_semantics=("parallel",)),
        name="ssm_scan")(x8, cw2, cb2, wp2, bp2, al4, ro4, rob4)

    ytok = jnp.take_along_axis(ysum.reshape(4, B, J, E), inv[:, None, :, None],
                               axis=2).reshape(4, M, E)

    MT4 = 256
    pf = pl.pallas_call(
        _back_kernel,
        grid=(4, M // MT4),
        in_specs=[pl.BlockSpec((1, MT4, E), lambda p, i: (p, i, 0)),
                  pl.BlockSpec((1, MT4, E), lambda p, i: (p, i, 0)),
                  pl.BlockSpec((MT4, D), lambda p, i: (i, 0)),
                  pl.BlockSpec((1, E, D), lambda p, i: (p, 0, 0)),
                  pl.BlockSpec((1, 1, D), lambda p, i: (p, 0, 0)),
                  pl.BlockSpec((D, PD), lambda p, i: (0, 0)),
                  pl.BlockSpec((1, PD), lambda p, i: (0, 0))],
        out_specs=pl.BlockSpec((1, MT4, PD), lambda p, i: (p, i, 0)),
        out_shape=jax.ShapeDtypeStruct((4, M, PD), jnp.bfloat16),
        compiler_params=pltpu.CompilerParams(
            dimension_semantics=("parallel", "parallel")),
        name="vim_back")(
            ytok, g4, tokens,
            params['po_W'].transpose(0, 2, 1), params['po_b'][:, None, :],
            params['out_W'].T.astype(jnp.bfloat16), params['out_b'][None, :])

    cat = (pf.reshape(4, B, HP, WP, C_IN, P, P)
           .transpose(1, 2, 5, 3, 6, 0, 4).reshape(B, H, W, CC))
    zrow1 = jnp.zeros((B, 1, W, CC), jnp.bfloat16)
    tops = jnp.concatenate([zrow1, cat[:, P - 1:H - 1:P]], axis=1)
    bots = jnp.concatenate([cat[:, P::P], zrow1], axis=1)
    halo = jnp.stack([tops, bots], axis=2).reshape(B * HP, 2, W, CC)

    wfd = params['fuse_W'].transpose(2, 3, 1, 0)
    wfr = jnp.zeros((3, CC, 384), jnp.float32)
    for dj in range(3):
        wfr = wfr.at[:, :, 128 * dj:128 * dj + C_IN].set(wfd[:, dj])
    wfr = wfr.astype(jnp.bfloat16)

    outc = pl.pallas_call(
        _conv_kernel,
        grid=(B * HP,),
        in_specs=[pl.BlockSpec((1, P * W, CC), lambda i: (i, 0, 0)),
                  pl.BlockSpec((1, 2 * W, CC), lambda i: (i, 0, 0)),
                  pl.BlockSpec((3, CC, 384), lambda i: (0, 0, 0)),
                  pl.BlockSpec((1, C_IN), lambda i: (0, 0))],
        out_specs=pl.BlockSpec((1, P * W, C_IN), lambda i: (i, 0, 0)),
        out_shape=jax.ShapeDtypeStruct((B * HP, P * W, C_IN), jnp.float32),
        compiler_params=pltpu.CompilerParams(dimension_semantics=("parallel",)),
        name="fuse_conv")(
            cat.reshape(B * HP, P * W, CC), halo.reshape(B * HP, 2 * W, CC),
            wfr, params['fuse_b'][None, :])

    return outc.reshape(B, H, W, C_IN).transpose(0, 3, 1, 2)
